# baseline (speedup 1.0000x reference)
import functools

import jax
import jax.numpy as jnp
from jax.experimental import pallas as pl
from jax.experimental.pallas import tpu as pltpu

_NC = 10

_INT_MIN = -2147483648


def _keys(s):
    b = jax.lax.bitcast_convert_type(s, jnp.int32)
    return jnp.where(b >= 0, b, jnp.int32(_INT_MIN) - b)


def _cc_kernel(scores_full_ref, scores_blk_ref, feat_ref, scale_ref, out_ref,
               keys_ref, thr_ref, m_ref, fbuf_ref, sems_ref, *,
               nsteps, br, k, npad, idx_bits, nbuf):
    step = pl.program_id(0)
    nc = out_ref.shape[0]
    kf = jnp.float32(k)

    def fcopy(slot, blk):
        return pltpu.make_async_copy(
            feat_ref.at[pl.ds(blk * br, br), :], fbuf_ref.at[slot],
            sems_ref.at[slot])

    @pl.when(step == 0)
    def _prologue():
        for b in range(nbuf):
            fcopy(b, b).start()

    @pl.when(step == 0)
    def _selection():
        keys_ref[...] = _keys(scores_full_ref[...])
        out_ref[...] = jnp.zeros_like(out_ref)

        def count_ge(cand):
            hit = keys_ref[...] >= cand
            return jnp.sum(jnp.where(hit, 1.0, 0.0), axis=1, keepdims=True)

        n_nonneg = count_ge(jnp.zeros((nc, 1), jnp.int32))
        prefix = jnp.where(n_nonneg >= kf,
                           jnp.zeros((nc, 1), jnp.int32),
                           jnp.full((nc, 1), _INT_MIN, jnp.int32))

        def tbody(i, p):
            bit = jax.lax.shift_left(jnp.int32(1), jnp.int32(30) - i)
            cand = p | bit
            return jnp.where(count_ge(cand) >= kf, cand, p)

        thr = jax.lax.fori_loop(0, 31, tbody, prefix)
        thr_ref[...] = thr

        keys = keys_ref[...]
        n_gt = jnp.sum(jnp.where(keys > thr, 1.0, 0.0), axis=1, keepdims=True)
        n_ge = jnp.sum(jnp.where(keys >= thr, 1.0, 0.0), axis=1, keepdims=True)
        r = kf - n_gt
        big = jnp.int32(1 << idx_bits)
        m0 = jnp.where(n_ge == kf, jnp.full((nc, 1), big),
                       jnp.zeros((nc, 1), jnp.int32))
        trips = jnp.where(jnp.any(n_ge != kf), idx_bits, 0)
        idx = jax.lax.broadcasted_iota(jnp.int32, (nc, npad), 1)

        def mbody(i, m):
            cand = m | jax.lax.shift_left(jnp.int32(1),
                                          jnp.int32(idx_bits - 1) - i)
            tied_below = (keys_ref[...] == thr) & (idx <= cand)
            cnt = jnp.sum(jnp.where(tied_below, 1.0, 0.0), axis=1,
                          keepdims=True)
            return jnp.where(cnt <= r, cand, m)

        m_ref[...] = jax.lax.fori_loop(0, trips, mbody, m0)

    slot = jax.lax.rem(step, nbuf)
    fcopy(slot, step).wait()
    kblk = _keys(scores_blk_ref[0])
    iblk = jax.lax.broadcasted_iota(jnp.int32, (nc, br), 1) + step * br
    thr = thr_ref[...]
    sel = (kblk > thr) | ((kblk == thr) & (iblk <= m_ref[...]))
    w = jnp.where(sel, 1.0, 0.0)
    out_ref[...] += jax.lax.dot_general(
        w, fbuf_ref[slot], (((1,), (0,)), ((), ())),
        preferred_element_type=jnp.float32)

    @pl.when(step + nbuf < nsteps)
    def _refill():
        fcopy(slot, step + nbuf).start()

    @pl.when(step == nsteps - 1)
    def _finish():
        out_ref[...] *= scale_ref[0, 0]


def kernel(image_features, image_scores, xi_c):
    n, d = image_features.shape
    nc = image_scores.shape[1]
    k = n // nc
    br = 2000
    nsteps = n // br
    npad = ((n + 127) // 128) * 128
    idx_bits = max(1, (npad - 1).bit_length())

    scores_t = image_scores.T
    scores_tp = jnp.pad(scores_t, ((0, 0), (0, npad - n)),
                        constant_values=-jnp.inf)
    scores_blocks = scores_t.reshape(nc, nsteps, br).transpose(1, 0, 2)

    topk_mask = (jnp.arange(k) < k * xi_c).astype(image_features.dtype)
    scale = (1.0 / jnp.sum(topk_mask)).astype(jnp.float32).reshape(1, 1)

    nbuf = 20
    body = functools.partial(_cc_kernel, nsteps=nsteps, br=br, k=k,
                             npad=npad, idx_bits=idx_bits, nbuf=nbuf)
    return pl.pallas_call(
        body,
        grid=(nsteps,),
        in_specs=[
            pl.BlockSpec((nc, npad), lambda j: (0, 0)),
            pl.BlockSpec((1, nc, br), lambda j: (j, 0, 0)),
            pl.BlockSpec(memory_space=pl.ANY),
            pl.BlockSpec(memory_space=pltpu.SMEM),
        ],
        out_specs=pl.BlockSpec((nc, d), lambda j: (0, 0)),
        out_shape=jax.ShapeDtypeStruct((nc, d), jnp.float32),
        scratch_shapes=[
            pltpu.VMEM((nc, npad), jnp.int32),
            pltpu.VMEM((nc, 1), jnp.int32),
            pltpu.VMEM((nc, 1), jnp.int32),
            pltpu.VMEM((nbuf, br, d), jnp.float32),
            pltpu.SemaphoreType.DMA((nbuf,)),
        ],
        compiler_params=pltpu.CompilerParams(
            dimension_semantics=("arbitrary",)),
    )(scores_tp, scores_blocks, image_features, scale)


# speedup vs baseline: 1.1735x; 1.1735x over previous
import functools

import jax
import jax.numpy as jnp
from jax.experimental import pallas as pl
from jax.experimental.pallas import tpu as pltpu

_NC = 10

_INT_MIN = -2147483648


def _keys(s):
    b = jax.lax.bitcast_convert_type(s, jnp.int32)
    return jnp.where(b >= 0, b, jnp.int32(_INT_MIN) - b)


def _cc_kernel(scores_full_ref, scores_blk_ref, feat_ref, scale_ref, out_ref,
               keys_ref, thr_ref, m_ref, fbuf_ref, sems_ref, *,
               nsteps, br, k, npad, idx_bits, nbuf):
    step = pl.program_id(0)
    nc = out_ref.shape[0]
    kf = jnp.float32(k)

    def fcopy(slot, blk):
        return pltpu.make_async_copy(
            feat_ref.at[pl.ds(blk * br, br), :], fbuf_ref.at[slot],
            sems_ref.at[slot])

    @pl.when(step == 0)
    def _prologue():
        for b in range(nbuf):
            fcopy(b, b).start()

    @pl.when(step == 0)
    def _selection():
        keys_ref[...] = _keys(scores_full_ref[...])
        out_ref[...] = jnp.zeros_like(out_ref)

        def count_ge(cand):
            hit = keys_ref[...] >= cand
            return jnp.sum(jnp.where(hit, 1.0, 0.0), axis=1, keepdims=True)

        n_nonneg = count_ge(jnp.zeros((nc, 1), jnp.int32))
        prefix = jnp.where(n_nonneg >= kf,
                           jnp.zeros((nc, 1), jnp.int32),
                           jnp.full((nc, 1), _INT_MIN, jnp.int32))

        def tbody(i, p):
            bit = jax.lax.shift_left(jnp.int32(1), jnp.int32(30) - i)
            cand = p | bit
            return jnp.where(count_ge(cand) >= kf, cand, p)

        thr = jax.lax.fori_loop(0, 31, tbody, prefix)
        thr_ref[...] = thr

        keys = keys_ref[...]
        n_gt = jnp.sum(jnp.where(keys > thr, 1.0, 0.0), axis=1, keepdims=True)
        n_ge = jnp.sum(jnp.where(keys >= thr, 1.0, 0.0), axis=1, keepdims=True)
        r = kf - n_gt
        big = jnp.int32(1 << idx_bits)
        m0 = jnp.where(n_ge == kf, jnp.full((nc, 1), big),
                       jnp.zeros((nc, 1), jnp.int32))
        trips = jnp.where(jnp.any(n_ge != kf), idx_bits, 0)
        idx = jax.lax.broadcasted_iota(jnp.int32, (nc, npad), 1)

        def mbody(i, m):
            cand = m | jax.lax.shift_left(jnp.int32(1),
                                          jnp.int32(idx_bits - 1) - i)
            tied_below = (keys_ref[...] == thr) & (idx <= cand)
            cnt = jnp.sum(jnp.where(tied_below, 1.0, 0.0), axis=1,
                          keepdims=True)
            return jnp.where(cnt <= r, cand, m)

        m_ref[...] = jax.lax.fori_loop(0, trips, mbody, m0)

    slot = jax.lax.rem(step, nbuf)
    fcopy(slot, step).wait()
    kblk = _keys(scores_blk_ref[0])
    iblk = jax.lax.broadcasted_iota(jnp.int32, (nc, br), 1) + step * br
    thr = thr_ref[...]
    sel = (kblk > thr) | ((kblk == thr) & (iblk <= m_ref[...]))
    w = jnp.where(sel, 1.0, 0.0)
    out_ref[...] += jax.lax.dot_general(
        w, fbuf_ref[slot], (((1,), (0,)), ((), ())),
        preferred_element_type=jnp.float32)

    @pl.when(step + nbuf < nsteps)
    def _refill():
        fcopy(slot, step + nbuf).start()

    @pl.when(step == nsteps - 1)
    def _finish():
        out_ref[...] *= scale_ref[0, 0]


def kernel(image_features, image_scores, xi_c):
    n, d = image_features.shape
    nc = image_scores.shape[1]
    k = n // nc
    br = 5000
    nsteps = n // br
    npad = ((n + 127) // 128) * 128
    idx_bits = max(1, (npad - 1).bit_length())

    scores_t = image_scores.T
    scores_tp = jnp.pad(scores_t, ((0, 0), (0, npad - n)),
                        constant_values=-jnp.inf)
    scores_blocks = scores_t.reshape(nc, nsteps, br).transpose(1, 0, 2)

    topk_mask = (jnp.arange(k) < k * xi_c).astype(image_features.dtype)
    scale = (1.0 / jnp.sum(topk_mask)).astype(jnp.float32).reshape(1, 1)

    nbuf = 8
    body = functools.partial(_cc_kernel, nsteps=nsteps, br=br, k=k,
                             npad=npad, idx_bits=idx_bits, nbuf=nbuf)
    return pl.pallas_call(
        body,
        grid=(nsteps,),
        in_specs=[
            pl.BlockSpec((nc, npad), lambda j: (0, 0)),
            pl.BlockSpec((1, nc, br), lambda j: (j, 0, 0)),
            pl.BlockSpec(memory_space=pl.ANY),
            pl.BlockSpec(memory_space=pltpu.SMEM),
        ],
        out_specs=pl.BlockSpec((nc, d), lambda j: (0, 0)),
        out_shape=jax.ShapeDtypeStruct((nc, d), jnp.float32),
        scratch_shapes=[
            pltpu.VMEM((nc, npad), jnp.int32),
            pltpu.VMEM((nc, 1), jnp.int32),
            pltpu.VMEM((nc, 1), jnp.int32),
            pltpu.VMEM((nbuf, br, d), jnp.float32),
            pltpu.SemaphoreType.DMA((nbuf,)),
        ],
        compiler_params=pltpu.CompilerParams(
            dimension_semantics=("arbitrary",)),
    )(scores_tp, scores_blocks, image_features, scale)


# speedup vs baseline: 1.1748x; 1.0011x over previous
import functools

import jax
import jax.numpy as jnp
from jax.experimental import pallas as pl
from jax.experimental.pallas import tpu as pltpu

_INT_MIN = -2147483648


def _keys(s):
    b = jax.lax.bitcast_convert_type(s, jnp.int32)
    return jnp.where(b >= 0, b, jnp.int32(_INT_MIN) - b)


def _cc_kernel(scores_full_ref, scores_blk_ref, feat_ref, scale_ref, out_ref,
               keys_ref, thr_ref, m_ref, fbuf_ref, sems_ref, *,
               nsteps, br, k, npad, idx_bits, nbuf):
    step = pl.program_id(0)
    nc = out_ref.shape[0]
    kf = jnp.float32(k)

    def fcopy(slot, blk):
        return pltpu.make_async_copy(
            feat_ref.at[pl.ds(blk * br, br), :], fbuf_ref.at[slot],
            sems_ref.at[slot])

    @pl.when(step == 0)
    def _prologue():
        for b in range(nbuf):
            fcopy(b, b).start()

    @pl.when(step == 0)
    def _selection():
        keys_ref[...] = _keys(scores_full_ref[...])
        out_ref[...] = jnp.zeros_like(out_ref)

        def count_ge(cand):
            hit = keys_ref[...] >= cand
            return jnp.sum(jnp.where(hit, 1.0, 0.0), axis=1, keepdims=True)

        n_nonneg = count_ge(jnp.zeros((nc, 1), jnp.int32))
        prefix = jnp.where(n_nonneg >= kf,
                           jnp.zeros((nc, 1), jnp.int32),
                           jnp.full((nc, 1), _INT_MIN, jnp.int32))

        def tbody(i, p):
            bit = jax.lax.shift_left(jnp.int32(1), jnp.int32(30) - i)
            cand = p | bit
            return jnp.where(count_ge(cand) >= kf, cand, p)

        thr = jax.lax.fori_loop(0, 31, tbody, prefix)
        thr_ref[...] = thr

        keys = keys_ref[...]
        n_gt = jnp.sum(jnp.where(keys > thr, 1.0, 0.0), axis=1, keepdims=True)
        n_ge = jnp.sum(jnp.where(keys >= thr, 1.0, 0.0), axis=1, keepdims=True)
        r = kf - n_gt
        big = jnp.int32(1 << idx_bits)
        m0 = jnp.where(n_ge == kf, jnp.full((nc, 1), big),
                       jnp.zeros((nc, 1), jnp.int32))
        trips = jnp.where(jnp.any(n_ge != kf), idx_bits, 0)
        idx = jax.lax.broadcasted_iota(jnp.int32, (nc, npad), 1)

        def mbody(i, m):
            cand = m | jax.lax.shift_left(jnp.int32(1),
                                          jnp.int32(idx_bits - 1) - i)
            tied_below = (keys_ref[...] == thr) & (idx <= cand)
            cnt = jnp.sum(jnp.where(tied_below, 1.0, 0.0), axis=1,
                          keepdims=True)
            return jnp.where(cnt <= r, cand, m)

        m_ref[...] = jax.lax.fori_loop(0, trips, mbody, m0)

    slot = jax.lax.rem(step, nbuf)
    fcopy(slot, step).wait()
    kblk = _keys(scores_blk_ref[0])
    iblk = jax.lax.broadcasted_iota(jnp.int32, (nc, br), 1) + step * br
    thr = thr_ref[...]
    sel = (kblk > thr) | ((kblk == thr) & (iblk <= m_ref[...]))
    w = jnp.where(sel, 1.0, 0.0)
    out_ref[...] += jax.lax.dot_general(
        w, fbuf_ref[slot], (((1,), (0,)), ((), ())),
        preferred_element_type=jnp.float32)

    @pl.when(step + nbuf < nsteps)
    def _refill():
        fcopy(slot, step + nbuf).start()

    @pl.when(step == nsteps - 1)
    def _finish():
        out_ref[...] *= scale_ref[0, 0]


def kernel(image_features, image_scores, xi_c):
    n, d = image_features.shape
    nc = image_scores.shape[1]
    k = n // nc
    br = 5000
    nsteps = n // br
    npad = ((n + 127) // 128) * 128
    idx_bits = max(1, (npad - 1).bit_length())

    scores_t = image_scores.T
    scores_tp = jnp.pad(scores_t, ((0, 0), (0, npad - n)),
                        constant_values=-jnp.inf)
    scores_blocks = scores_t.reshape(nc, nsteps, br).transpose(1, 0, 2)

    topk_mask = (jnp.arange(k) < k * xi_c).astype(image_features.dtype)
    scale = (1.0 / jnp.sum(topk_mask)).astype(jnp.float32).reshape(1, 1)

    nbuf = 8
    body = functools.partial(_cc_kernel, nsteps=nsteps, br=br, k=k,
                             npad=npad, idx_bits=idx_bits, nbuf=nbuf)
    return pl.pallas_call(
        body,
        grid=(nsteps,),
        in_specs=[
            pl.BlockSpec((nc, npad), lambda j: (0, 0)),
            pl.BlockSpec((1, nc, br), lambda j: (j, 0, 0)),
            pl.BlockSpec(memory_space=pl.ANY),
            pl.BlockSpec(memory_space=pltpu.SMEM),
        ],
        out_specs=pl.BlockSpec((nc, d), lambda j: (0, 0)),
        out_shape=jax.ShapeDtypeStruct((nc, d), jnp.float32),
        scratch_shapes=[
            pltpu.VMEM((nc, npad), jnp.int32),
            pltpu.VMEM((nc, 1), jnp.int32),
            pltpu.VMEM((nc, 1), jnp.int32),
            pltpu.VMEM((nbuf, br, d), jnp.float32),
            pltpu.SemaphoreType.DMA((nbuf,)),
        ],
        compiler_params=pltpu.CompilerParams(
            dimension_semantics=("arbitrary",)),
    )(scores_tp, scores_blocks, image_features, scale)
